# baseline (speedup 1.0000x reference)
import functools

import jax
import jax.numpy as jnp
from jax import lax
from jax.experimental import pallas as pl
from jax.experimental.pallas import tpu as pltpu
from jax.experimental.pallas import tpu_sc as plsc

N = 50000
E = 800000
DN = 64
H = 64

_NC = 2
_NS = 16
_NW = _NC * _NS

_XW_BLK = 2000


_TDOT = (((0,), (0,)), ((), ()))


def _xw_body(x_ref, w_ref, b_ref, o_ref):
    o_ref[...] = (
        jnp.dot(x_ref[...], w_ref[...], preferred_element_type=jnp.float32)
        + b_ref[...]
    )


def _tc_xw(x, w_top, b1a):
    return pl.pallas_call(
        _xw_body,
        grid=(N // _XW_BLK,),
        in_specs=[
            pl.BlockSpec((_XW_BLK, DN), lambda i: (i, 0)),
            pl.BlockSpec((DN, H), lambda i: (0, 0)),
            pl.BlockSpec((1, H), lambda i: (0, 0)),
        ],
        out_specs=pl.BlockSpec((_XW_BLK, H), lambda i: (i, 0)),
        out_shape=jax.ShapeDtypeStruct((N, H), jnp.float32),
    )(x, w_top, b1a)


_EPW = E // _NW
_GCH = 1000
_GSUB = (128, 128, 128, 128, 128, 128, 128, 104)


def _sc_gather_body(xw_hbm, row_hbm, g_hbm, idx_v, rows_v, sem):
    c = lax.axis_index("c")
    s = lax.axis_index("s")
    wid = s * _NC + c
    base = wid * _EPW

    def chunk(k, carry):
        off = base + k * _GCH
        pltpu.sync_copy(row_hbm.at[pl.ds(off, _GCH)], idx_v)
        descs = []
        o = 0
        for sz in _GSUB:
            descs.append(
                pltpu.async_copy(
                    xw_hbm.at[idx_v.at[pl.ds(o, sz)]],
                    rows_v.at[pl.ds(o, sz)],
                    sem,
                )
            )
            o += sz
        for d in descs:
            d.wait()
        pltpu.sync_copy(rows_v, g_hbm.at[pl.ds(off, _GCH)])
        return carry

    lax.fori_loop(0, _EPW // _GCH, chunk, 0)


def _sc_gather(xw, row):
    mesh = plsc.VectorSubcoreMesh(core_axis_name="c", subcore_axis_name="s")
    f = pl.kernel(
        _sc_gather_body,
        out_type=jax.ShapeDtypeStruct((E, H), jnp.float32),
        mesh=mesh,
        scratch_types=[
            pltpu.VMEM((_GCH,), jnp.int32),
            pltpu.VMEM((_GCH, H), jnp.float32),
            pltpu.SemaphoreType.DMA,
        ],
        compiler_params=pltpu.CompilerParams(use_tc_tiling_on_sc=False),
    )
    return f(xw, row)


_E_BLK = 3200


_HQ = H // 4


def _edge_body(g_ref, ea_ref, w_ref, h0_ref, h1_ref, h2_ref, h3_ref,
               sh_ref, sh2_ref):
    z = g_ref[...] + jnp.dot(
        ea_ref[...], w_ref[...], preferred_element_type=jnp.float32
    )
    h = jnp.maximum(z, 0.01 * z)
    h0_ref[...] = h[:, 0 * _HQ : 1 * _HQ]
    h1_ref[...] = h[:, 1 * _HQ : 2 * _HQ]
    h2_ref[...] = h[:, 2 * _HQ : 3 * _HQ]
    h3_ref[...] = h[:, 3 * _HQ : 4 * _HQ]

    @pl.when(pl.program_id(0) == 0)
    def _():
        sh_ref[...] = jnp.zeros_like(sh_ref)
        sh2_ref[...] = jnp.zeros_like(sh2_ref)

    sh_ref[...] += jnp.sum(h, axis=0, keepdims=True)
    sh2_ref[...] += jnp.sum(h * h, axis=0, keepdims=True)


def _tc_edge(g, edge_attr, w_bot):
    hspec = pl.BlockSpec((_E_BLK, _HQ), lambda i: (i, 0))
    hshape = jax.ShapeDtypeStruct((E, _HQ), jnp.float32)
    return pl.pallas_call(
        _edge_body,
        grid=(E // _E_BLK,),
        in_specs=[
            pl.BlockSpec((_E_BLK, H), lambda i: (i, 0)),
            pl.BlockSpec((_E_BLK, DN), lambda i: (i, 0)),
            pl.BlockSpec((DN, H), lambda i: (0, 0)),
        ],
        out_specs=[
            hspec, hspec, hspec, hspec,
            pl.BlockSpec((1, H), lambda i: (0, 0)),
            pl.BlockSpec((1, H), lambda i: (0, 0)),
        ],
        out_shape=[
            hshape, hshape, hshape, hshape,
            jax.ShapeDtypeStruct((1, H), jnp.float32),
            jax.ShapeDtypeStruct((1, H), jnp.float32),
        ],
    )(g, edge_attr, w_bot)


_EPT = E // _NS
_IW = 125
_IR = 8
_SCH = _IW * _IR
_NST = 2
_NPT = N // _NS
_ZCH = 125
_DW = 8


def _sc_scatter_body(
    h0_hbm, h1_hbm, h2_hbm, h3_hbm, col2_hbm, z16_hbm,
    s0_hbm, s1_hbm, s2_hbm, s3_hbm,
    idx2_v, h_v, zb_v, sem, sacc,
):
    c = lax.axis_index("c")
    s = lax.axis_index("s")
    base = s * _EPT
    ibase = s * (_EPT // _SCH)

    def _scan(h_hbm):
        def chunk(k, carry):
            pltpu.sync_copy(col2_hbm.at[ibase + k], idx2_v)
            pltpu.sync_copy(h_hbm.at[pl.ds(base + k * _SCH, _SCH)], h_v)
            descs = [
                pltpu.async_copy(
                    h_v.at[pl.ds(j * _IW, _IW)],
                    sacc.at[idx2_v.at[j]],
                    sem,
                    add=True,
                )
                for j in range(_IR)
            ]
            for d in descs:
                d.wait()
            return carry

        lax.fori_loop(0, _EPT // _SCH, chunk, 0)

    def _dump(s_hbm):
        for k in range(_NPT // _ZCH):
            r = s * _NPT + k * _ZCH
            pltpu.sync_copy(sacc.at[pl.ds(r, _ZCH)], zb_v)
            pltpu.sync_copy(zb_v, s_hbm.at[pl.ds(r, _ZCH)])

    for p, (hA, hB, sA, sB) in enumerate(
        ((h0_hbm, h2_hbm, s0_hbm, s2_hbm), (h1_hbm, h3_hbm, s1_hbm, s3_hbm))
    ):
        for k in range(_NPT // _ZCH):
            r = s * _NPT + k * _ZCH
            pltpu.sync_copy(z16_hbm.at[pl.ds(r, _ZCH)], zb_v)
            pltpu.sync_copy(zb_v, sacc.at[pl.ds(r, _ZCH)])

        plsc.subcore_barrier()

        @pl.when(c == 0)
        def _():
            _scan(hA)

        @pl.when(c == 1)
        def _():
            _scan(hB)

        plsc.subcore_barrier()

        @pl.when(c == 0)
        def _():
            _dump(sA)

        @pl.when(c == 1)
        def _():
            _dump(sB)


def _sc_scatter(h0, h1, h2, h3, col2):
    mesh = plsc.VectorSubcoreMesh(core_axis_name="c", subcore_axis_name="s")
    z16 = jnp.zeros((N, _HQ), jnp.float32)
    sshape = jax.ShapeDtypeStruct((N, _HQ), jnp.float32)
    f = pl.kernel(
        _sc_scatter_body,
        out_type=(sshape, sshape, sshape, sshape),
        mesh=mesh,
        scratch_types=[
            pltpu.VMEM((_IR, _IW), jnp.int32),
            pltpu.VMEM((_SCH, _HQ), jnp.float32),
            pltpu.VMEM((_ZCH, _HQ), jnp.float32),
            pltpu.SemaphoreType.DMA,
            pltpu.VMEM_SHARED((N, _HQ), jnp.float32),
        ],
        compiler_params=pltpu.CompilerParams(use_tc_tiling_on_sc=False),
    )
    return f(h0, h1, h2, h3, col2, z16)


def _sc_deg_body(
    col2_hbm, zd_hbm, ones_hbm,
    d0_hbm, d1_hbm,
    idx2_v, ones_v, zbd_v, sem, dacc,
):
    c = lax.axis_index("c")
    s = lax.axis_index("s")

    for k in range(_NPT // _ZCH):
        r = s * _NPT + k * _ZCH
        pltpu.sync_copy(zd_hbm.at[pl.ds(r, _ZCH)], zbd_v)
        pltpu.sync_copy(zbd_v, dacc.at[pl.ds(r, _ZCH)])

    pltpu.sync_copy(ones_hbm, ones_v)
    plsc.subcore_barrier()

    wid = s * _NC + c
    ibase = wid * (_EPW // _SCH)

    def chunk(k, carry):
        pltpu.sync_copy(col2_hbm.at[ibase + k], idx2_v)
        descs = [
            pltpu.async_copy(ones_v, dacc.at[idx2_v.at[j]], sem, add=True)
            for j in range(_IR)
        ]
        for d in descs:
            d.wait()
        return carry

    lax.fori_loop(0, _EPW // _SCH, chunk, 0)
    plsc.subcore_barrier()

    def _dump(d_hbm):
        for k in range(_NPT // _ZCH):
            r = s * _NPT + k * _ZCH
            pltpu.sync_copy(dacc.at[pl.ds(r, _ZCH)], zbd_v)
            pltpu.sync_copy(zbd_v, d_hbm.at[pl.ds(r, _ZCH)])

    @pl.when(c == 0)
    def _():
        _dump(d0_hbm)

    @pl.when(c == 1)
    def _():
        _dump(d1_hbm)


def _sc_deg(col2):
    mesh = plsc.VectorSubcoreMesh(core_axis_name="c", subcore_axis_name="s")
    zd = jnp.zeros((N, _DW), jnp.float32)
    ones = jnp.ones((_IW, _DW), jnp.float32)
    f = pl.kernel(
        _sc_deg_body,
        out_type=(
            jax.ShapeDtypeStruct((N, _DW), jnp.float32),
            jax.ShapeDtypeStruct((N, _DW), jnp.float32),
        ),
        mesh=mesh,
        scratch_types=[
            pltpu.VMEM((_IR, _IW), jnp.int32),
            pltpu.VMEM((_IW, _DW), jnp.float32),
            pltpu.VMEM((_ZCH, _DW), jnp.float32),
            pltpu.SemaphoreType.DMA,
            pltpu.VMEM_SHARED((N, _DW), jnp.float32),
        ],
        compiler_params=pltpu.CompilerParams(use_tc_tiling_on_sc=False),
    )
    return f(col2, zd, ones)


_N_BLK = 2000


def _node1_body(
    s0_ref, s1_ref, s2_ref, s3_ref, d0_ref, d1_ref, x_ref, sh_ref, sh2_ref,
    g1_ref, be1_ref, w1b_ref, b1b_ref, w2at_ref, w2ab_ref, b2a_ref,
    t_ref, st_ref, st2_ref,
):
    mu = sh_ref[...] / E
    var = sh2_ref[...] / E - mu * mu
    winv = g1_ref[...] * lax.rsqrt(var + 1e-5)
    cvec = (
        jnp.dot(be1_ref[...] - winv * mu, w1b_ref[...],
                preferred_element_type=jnp.float32)
        + b1b_ref[...]
    )
    sb = jnp.concatenate(
        [s0_ref[...], s1_ref[...], s2_ref[...], s3_ref[...]], axis=1
    )
    deg = (d0_ref[...] + d1_ref[...])[:, :1]
    agg = (
        jnp.dot(winv * sb, w1b_ref[...], preferred_element_type=jnp.float32)
        + deg * cvec
    )
    t = (
        jnp.dot(x_ref[...], w2at_ref[...], preferred_element_type=jnp.float32)
        + jnp.dot(agg, w2ab_ref[...], preferred_element_type=jnp.float32)
        + b2a_ref[...]
    )
    t = jnp.maximum(t, 0.01 * t)
    t_ref[...] = t

    @pl.when(pl.program_id(0) == 0)
    def _():
        st_ref[...] = jnp.zeros_like(st_ref)
        st2_ref[...] = jnp.zeros_like(st2_ref)

    st_ref[...] += jnp.sum(t, axis=0, keepdims=True)
    st2_ref[...] += jnp.sum(t * t, axis=0, keepdims=True)


def _tc_node1(s0, s1, s2, s3, d0, d1, x, sh, sh2, g1, be1, w1b, b1b,
              w2at, w2ab, b2a):
    sspec = pl.BlockSpec((_N_BLK, _HQ), lambda i: (i, 0))
    return pl.pallas_call(
        _node1_body,
        grid=(N // _N_BLK,),
        in_specs=[
            sspec, sspec, sspec, sspec,
            pl.BlockSpec((_N_BLK, _DW), lambda i: (i, 0)),
            pl.BlockSpec((_N_BLK, _DW), lambda i: (i, 0)),
            pl.BlockSpec((_N_BLK, DN), lambda i: (i, 0)),
            pl.BlockSpec((1, H), lambda i: (0, 0)),
            pl.BlockSpec((1, H), lambda i: (0, 0)),
            pl.BlockSpec((1, H), lambda i: (0, 0)),
            pl.BlockSpec((1, H), lambda i: (0, 0)),
            pl.BlockSpec((H, H), lambda i: (0, 0)),
            pl.BlockSpec((1, H), lambda i: (0, 0)),
            pl.BlockSpec((DN, H), lambda i: (0, 0)),
            pl.BlockSpec((H, H), lambda i: (0, 0)),
            pl.BlockSpec((1, H), lambda i: (0, 0)),
        ],
        out_specs=[
            pl.BlockSpec((_N_BLK, H), lambda i: (i, 0)),
            pl.BlockSpec((1, H), lambda i: (0, 0)),
            pl.BlockSpec((1, H), lambda i: (0, 0)),
        ],
        out_shape=[
            jax.ShapeDtypeStruct((N, H), jnp.float32),
            jax.ShapeDtypeStruct((1, H), jnp.float32),
            jax.ShapeDtypeStruct((1, H), jnp.float32),
        ],
    )(s0, s1, s2, s3, d0, d1, x, sh, sh2, g1, be1, w1b, b1b, w2at, w2ab, b2a)


def _node2_body(t_ref, st_ref, st2_ref, g2_ref, be2_ref, w2b_ref, b2b_ref, o_ref):
    mu = st_ref[...] / N
    var = st2_ref[...] / N - mu * mu
    winv = g2_ref[...] * lax.rsqrt(var + 1e-5)
    y = winv * (t_ref[...] - mu) + be2_ref[...]
    o_ref[...] = (
        jnp.dot(y, w2b_ref[...], preferred_element_type=jnp.float32)
        + b2b_ref[...]
    )


def _tc_node2(t, st, st2, g2, be2, w2b, b2b):
    return pl.pallas_call(
        _node2_body,
        grid=(N // _N_BLK,),
        in_specs=[
            pl.BlockSpec((_N_BLK, H), lambda i: (i, 0)),
            pl.BlockSpec((1, H), lambda i: (0, 0)),
            pl.BlockSpec((1, H), lambda i: (0, 0)),
            pl.BlockSpec((1, H), lambda i: (0, 0)),
            pl.BlockSpec((1, H), lambda i: (0, 0)),
            pl.BlockSpec((H, DN), lambda i: (0, 0)),
            pl.BlockSpec((1, DN), lambda i: (0, 0)),
        ],
        out_specs=pl.BlockSpec((_N_BLK, DN), lambda i: (i, 0)),
        out_shape=jax.ShapeDtypeStruct((N, DN), jnp.float32),
    )(t, st, st2, g2, be2, w2b, b2b)


def kernel(x, edge_index, edge_attr, u, batch, W1a, b1a, g1, be1, W1b, b1b,
           W2a, b2a, g2, be2, W2b, b2b):
    row = edge_index[0]
    col2 = edge_index[1].reshape(E // _SCH, _IR, _IW)
    r1 = lambda v: v.reshape(1, -1)

    xw = _tc_xw(x, W1a[:DN], r1(b1a))
    g = _sc_gather(xw, row)
    h0, h1, h2, h3, sh, sh2 = _tc_edge(g, edge_attr, W1a[DN:])
    s0, s1, s2, s3 = _sc_scatter(h0, h1, h2, h3, col2)
    d0, d1 = _sc_deg(col2)
    t, st, st2 = _tc_node1(
        s0, s1, s2, s3, d0, d1, x, sh, sh2, r1(g1), r1(be1), W1b, r1(b1b),
        W2a[:DN], W2a[DN:], r1(b2a),
    )
    return _tc_node2(t, st, st2, r1(g2), r1(be2), W2b, r1(b2b))


# speedup vs baseline: 1.3239x; 1.3239x over previous
import functools

import jax
import jax.numpy as jnp
from jax import lax
from jax.experimental import pallas as pl
from jax.experimental.pallas import tpu as pltpu
from jax.experimental.pallas import tpu_sc as plsc

N = 50000
E = 800000
DN = 64
H = 64

_NC = 2
_NS = 16
_NW = _NC * _NS

_XW_BLK = 2000


_TDOT = (((0,), (0,)), ((), ()))


def _xw_body(x_ref, w_ref, b_ref, o_ref):
    o_ref[...] = (
        jnp.dot(x_ref[...], w_ref[...], preferred_element_type=jnp.float32)
        + b_ref[...]
    )


def _tc_xw(x, w_top, b1a):
    return pl.pallas_call(
        _xw_body,
        grid=(N // _XW_BLK,),
        in_specs=[
            pl.BlockSpec((_XW_BLK, DN), lambda i: (i, 0)),
            pl.BlockSpec((DN, H), lambda i: (0, 0)),
            pl.BlockSpec((1, H), lambda i: (0, 0)),
        ],
        out_specs=pl.BlockSpec((_XW_BLK, H), lambda i: (i, 0)),
        out_shape=jax.ShapeDtypeStruct((N, H), jnp.float32),
    )(x, w_top, b1a)


_EPW = E // _NW
_GCH = 1000
_GSUB = (128, 128, 128, 128, 128, 128, 128, 104)


def _sc_gather_body(xw_hbm, row_hbm, g_hbm, idx_v, rows_v, sem):
    c = lax.axis_index("c")
    s = lax.axis_index("s")
    wid = s * _NC + c
    base = wid * _EPW

    def chunk(k, carry):
        off = base + k * _GCH
        pltpu.sync_copy(row_hbm.at[pl.ds(off, _GCH)], idx_v)
        descs = []
        o = 0
        for sz in _GSUB:
            descs.append(
                pltpu.async_copy(
                    xw_hbm.at[idx_v.at[pl.ds(o, sz)]],
                    rows_v.at[pl.ds(o, sz)],
                    sem,
                )
            )
            o += sz
        for d in descs:
            d.wait()
        pltpu.sync_copy(rows_v, g_hbm.at[pl.ds(off, _GCH)])
        return carry

    lax.fori_loop(0, _EPW // _GCH, chunk, 0)


def _sc_gather(xw, row):
    mesh = plsc.VectorSubcoreMesh(core_axis_name="c", subcore_axis_name="s")
    f = pl.kernel(
        _sc_gather_body,
        out_type=jax.ShapeDtypeStruct((E, H), jnp.float32),
        mesh=mesh,
        scratch_types=[
            pltpu.VMEM((_GCH,), jnp.int32),
            pltpu.VMEM((_GCH, H), jnp.float32),
            pltpu.SemaphoreType.DMA,
        ],
        compiler_params=pltpu.CompilerParams(use_tc_tiling_on_sc=False),
    )
    return f(xw, row)


_E_BLK = 3200


_HH = H // 2


def _edge_body(g_ref, ea_ref, w_ref, h0_ref, h1_ref, sh_ref, sh2_ref):
    z = g_ref[...] + jnp.dot(
        ea_ref[...], w_ref[...], preferred_element_type=jnp.float32
    )
    h = jnp.maximum(z, 0.01 * z)
    h0_ref[...] = h[:, :_HH]
    h1_ref[...] = h[:, _HH:]

    @pl.when(pl.program_id(0) == 0)
    def _():
        sh_ref[...] = jnp.zeros_like(sh_ref)
        sh2_ref[...] = jnp.zeros_like(sh2_ref)

    sh_ref[...] += jnp.sum(h, axis=0, keepdims=True)
    sh2_ref[...] += jnp.sum(h * h, axis=0, keepdims=True)


def _tc_edge(g, edge_attr, w_bot):
    hspec = pl.BlockSpec((_E_BLK, _HH), lambda i: (i, 0))
    hshape = jax.ShapeDtypeStruct((E, _HH), jnp.float32)
    return pl.pallas_call(
        _edge_body,
        grid=(E // _E_BLK,),
        in_specs=[
            pl.BlockSpec((_E_BLK, H), lambda i: (i, 0)),
            pl.BlockSpec((_E_BLK, DN), lambda i: (i, 0)),
            pl.BlockSpec((DN, H), lambda i: (0, 0)),
        ],
        out_specs=[
            hspec, hspec,
            pl.BlockSpec((1, H), lambda i: (0, 0)),
            pl.BlockSpec((1, H), lambda i: (0, 0)),
        ],
        out_shape=[
            hshape, hshape,
            jax.ShapeDtypeStruct((1, H), jnp.float32),
            jax.ShapeDtypeStruct((1, H), jnp.float32),
        ],
    )(g, edge_attr, w_bot)


_EPT = E // _NS
_IW = 125
_IR = 4
_SCH = _IW * _IR
_NST = 2
_NPT = N // _NS
_ZCH = 125
_DW = 8


def _sc_scatter_body(
    h0_hbm, h1_hbm, col2_hbm, z32_hbm,
    s0_hbm, s1_hbm,
    idx2_v, h_v, zb_v, sem, sacc,
):
    c = lax.axis_index("c")
    s = lax.axis_index("s")
    base = s * _EPT
    ibase = s * (_EPT // _SCH)

    def _scan(h_hbm):
        def chunk(k, carry):
            pltpu.sync_copy(col2_hbm.at[ibase + k], idx2_v)
            pltpu.sync_copy(h_hbm.at[pl.ds(base + k * _SCH, _SCH)], h_v)
            descs = [
                pltpu.async_copy(
                    h_v.at[pl.ds(j * _IW, _IW)],
                    sacc.at[idx2_v.at[j]],
                    sem,
                    add=True,
                )
                for j in range(_IR)
            ]
            for d in descs:
                d.wait()
            return carry

        lax.fori_loop(0, _EPT // _SCH, chunk, 0)

    def _dump(s_hbm):
        for k in range(_NPT // _ZCH):
            r = s * _NPT + k * _ZCH
            pltpu.sync_copy(sacc.at[pl.ds(r, _ZCH)], zb_v)
            pltpu.sync_copy(zb_v, s_hbm.at[pl.ds(r, _ZCH)])

    for k in range(_NPT // _ZCH):
        r = s * _NPT + k * _ZCH
        pltpu.sync_copy(z32_hbm.at[pl.ds(r, _ZCH)], zb_v)
        pltpu.sync_copy(zb_v, sacc.at[pl.ds(r, _ZCH)])

    plsc.subcore_barrier()

    @pl.when(c == 0)
    def _():
        _scan(h0_hbm)

    @pl.when(c == 1)
    def _():
        _scan(h1_hbm)

    plsc.subcore_barrier()

    @pl.when(c == 0)
    def _():
        _dump(s0_hbm)

    @pl.when(c == 1)
    def _():
        _dump(s1_hbm)


def _sc_scatter(h0, h1, col2):
    mesh = plsc.VectorSubcoreMesh(core_axis_name="c", subcore_axis_name="s")
    z32 = jnp.zeros((N, _HH), jnp.float32)
    sshape = jax.ShapeDtypeStruct((N, _HH), jnp.float32)
    f = pl.kernel(
        _sc_scatter_body,
        out_type=(sshape, sshape),
        mesh=mesh,
        scratch_types=[
            pltpu.VMEM((_IR, _IW), jnp.int32),
            pltpu.VMEM((_SCH, _HH), jnp.float32),
            pltpu.VMEM((_ZCH, _HH), jnp.float32),
            pltpu.SemaphoreType.DMA,
            pltpu.VMEM_SHARED((N, _HH), jnp.float32),
        ],
        compiler_params=pltpu.CompilerParams(use_tc_tiling_on_sc=False),
    )
    return f(h0, h1, col2, z32)


def _sc_deg_body(
    col2_hbm, zd_hbm, ones_hbm,
    d0_hbm, d1_hbm,
    idx2_v, ones_v, zbd_v, sem, dacc,
):
    c = lax.axis_index("c")
    s = lax.axis_index("s")

    for k in range(_NPT // _ZCH):
        r = s * _NPT + k * _ZCH
        pltpu.sync_copy(zd_hbm.at[pl.ds(r, _ZCH)], zbd_v)
        pltpu.sync_copy(zbd_v, dacc.at[pl.ds(r, _ZCH)])

    pltpu.sync_copy(ones_hbm, ones_v)
    plsc.subcore_barrier()

    wid = s * _NC + c
    ibase = wid * (_EPW // _SCH)

    def chunk(k, carry):
        pltpu.sync_copy(col2_hbm.at[ibase + k], idx2_v)
        descs = [
            pltpu.async_copy(ones_v, dacc.at[idx2_v.at[j]], sem, add=True)
            for j in range(_IR)
        ]
        for d in descs:
            d.wait()
        return carry

    lax.fori_loop(0, _EPW // _SCH, chunk, 0)
    plsc.subcore_barrier()

    def _dump(d_hbm):
        for k in range(_NPT // _ZCH):
            r = s * _NPT + k * _ZCH
            pltpu.sync_copy(dacc.at[pl.ds(r, _ZCH)], zbd_v)
            pltpu.sync_copy(zbd_v, d_hbm.at[pl.ds(r, _ZCH)])

    @pl.when(c == 0)
    def _():
        _dump(d0_hbm)

    @pl.when(c == 1)
    def _():
        _dump(d1_hbm)


def _sc_deg(col2):
    mesh = plsc.VectorSubcoreMesh(core_axis_name="c", subcore_axis_name="s")
    zd = jnp.zeros((N, _DW), jnp.float32)
    ones = jnp.ones((_IW, _DW), jnp.float32)
    f = pl.kernel(
        _sc_deg_body,
        out_type=(
            jax.ShapeDtypeStruct((N, _DW), jnp.float32),
            jax.ShapeDtypeStruct((N, _DW), jnp.float32),
        ),
        mesh=mesh,
        scratch_types=[
            pltpu.VMEM((_IR, _IW), jnp.int32),
            pltpu.VMEM((_IW, _DW), jnp.float32),
            pltpu.VMEM((_ZCH, _DW), jnp.float32),
            pltpu.SemaphoreType.DMA,
            pltpu.VMEM_SHARED((N, _DW), jnp.float32),
        ],
        compiler_params=pltpu.CompilerParams(use_tc_tiling_on_sc=False),
    )
    return f(col2, zd, ones)


_N_BLK = 2000


def _node1_body(
    s0_ref, s1_ref, d0_ref, d1_ref, x_ref, sh_ref, sh2_ref,
    g1_ref, be1_ref, w1b_ref, b1b_ref, w2at_ref, w2ab_ref, b2a_ref,
    t_ref, st_ref, st2_ref,
):
    mu = sh_ref[...] / E
    var = sh2_ref[...] / E - mu * mu
    winv = g1_ref[...] * lax.rsqrt(var + 1e-5)
    cvec = (
        jnp.dot(be1_ref[...] - winv * mu, w1b_ref[...],
                preferred_element_type=jnp.float32)
        + b1b_ref[...]
    )
    sb = jnp.concatenate([s0_ref[...], s1_ref[...]], axis=1)
    deg = (d0_ref[...] + d1_ref[...])[:, :1]
    agg = (
        jnp.dot(winv * sb, w1b_ref[...], preferred_element_type=jnp.float32)
        + deg * cvec
    )
    t = (
        jnp.dot(x_ref[...], w2at_ref[...], preferred_element_type=jnp.float32)
        + jnp.dot(agg, w2ab_ref[...], preferred_element_type=jnp.float32)
        + b2a_ref[...]
    )
    t = jnp.maximum(t, 0.01 * t)
    t_ref[...] = t

    @pl.when(pl.program_id(0) == 0)
    def _():
        st_ref[...] = jnp.zeros_like(st_ref)
        st2_ref[...] = jnp.zeros_like(st2_ref)

    st_ref[...] += jnp.sum(t, axis=0, keepdims=True)
    st2_ref[...] += jnp.sum(t * t, axis=0, keepdims=True)


def _tc_node1(s0, s1, d0, d1, x, sh, sh2, g1, be1, w1b, b1b,
              w2at, w2ab, b2a):
    sspec = pl.BlockSpec((_N_BLK, _HH), lambda i: (i, 0))
    return pl.pallas_call(
        _node1_body,
        grid=(N // _N_BLK,),
        in_specs=[
            sspec, sspec,
            pl.BlockSpec((_N_BLK, _DW), lambda i: (i, 0)),
            pl.BlockSpec((_N_BLK, _DW), lambda i: (i, 0)),
            pl.BlockSpec((_N_BLK, DN), lambda i: (i, 0)),
            pl.BlockSpec((1, H), lambda i: (0, 0)),
            pl.BlockSpec((1, H), lambda i: (0, 0)),
            pl.BlockSpec((1, H), lambda i: (0, 0)),
            pl.BlockSpec((1, H), lambda i: (0, 0)),
            pl.BlockSpec((H, H), lambda i: (0, 0)),
            pl.BlockSpec((1, H), lambda i: (0, 0)),
            pl.BlockSpec((DN, H), lambda i: (0, 0)),
            pl.BlockSpec((H, H), lambda i: (0, 0)),
            pl.BlockSpec((1, H), lambda i: (0, 0)),
        ],
        out_specs=[
            pl.BlockSpec((_N_BLK, H), lambda i: (i, 0)),
            pl.BlockSpec((1, H), lambda i: (0, 0)),
            pl.BlockSpec((1, H), lambda i: (0, 0)),
        ],
        out_shape=[
            jax.ShapeDtypeStruct((N, H), jnp.float32),
            jax.ShapeDtypeStruct((1, H), jnp.float32),
            jax.ShapeDtypeStruct((1, H), jnp.float32),
        ],
    )(s0, s1, d0, d1, x, sh, sh2, g1, be1, w1b, b1b, w2at, w2ab, b2a)


def _node2_body(t_ref, st_ref, st2_ref, g2_ref, be2_ref, w2b_ref, b2b_ref, o_ref):
    mu = st_ref[...] / N
    var = st2_ref[...] / N - mu * mu
    winv = g2_ref[...] * lax.rsqrt(var + 1e-5)
    y = winv * (t_ref[...] - mu) + be2_ref[...]
    o_ref[...] = (
        jnp.dot(y, w2b_ref[...], preferred_element_type=jnp.float32)
        + b2b_ref[...]
    )


def _tc_node2(t, st, st2, g2, be2, w2b, b2b):
    return pl.pallas_call(
        _node2_body,
        grid=(N // _N_BLK,),
        in_specs=[
            pl.BlockSpec((_N_BLK, H), lambda i: (i, 0)),
            pl.BlockSpec((1, H), lambda i: (0, 0)),
            pl.BlockSpec((1, H), lambda i: (0, 0)),
            pl.BlockSpec((1, H), lambda i: (0, 0)),
            pl.BlockSpec((1, H), lambda i: (0, 0)),
            pl.BlockSpec((H, DN), lambda i: (0, 0)),
            pl.BlockSpec((1, DN), lambda i: (0, 0)),
        ],
        out_specs=pl.BlockSpec((_N_BLK, DN), lambda i: (i, 0)),
        out_shape=jax.ShapeDtypeStruct((N, DN), jnp.float32),
    )(t, st, st2, g2, be2, w2b, b2b)


def kernel(x, edge_index, edge_attr, u, batch, W1a, b1a, g1, be1, W1b, b1b,
           W2a, b2a, g2, be2, W2b, b2b):
    row = edge_index[0]
    col2 = edge_index[1].reshape(E // _SCH, _IR, _IW)
    r1 = lambda v: v.reshape(1, -1)

    xw = _tc_xw(x, W1a[:DN], r1(b1a))
    g = _sc_gather(xw, row)
    h0, h1, sh, sh2 = _tc_edge(g, edge_attr, W1a[DN:])
    s0, s1 = _sc_scatter(h0, h1, col2)
    d0, d1 = _sc_deg(col2)
    t, st, st2 = _tc_node1(
        s0, s1, d0, d1, x, sh, sh2, r1(g1), r1(be1), W1b, r1(b1b),
        W2a[:DN], W2a[DN:], r1(b2a),
    )
    return _tc_node2(t, st, st2, r1(g2), r1(be2), W2b, r1(b2b))


# speedup vs baseline: 1.3420x; 1.0137x over previous
import functools

import jax
import jax.numpy as jnp
from jax import lax
from jax.experimental import pallas as pl
from jax.experimental.pallas import tpu as pltpu
from jax.experimental.pallas import tpu_sc as plsc

N = 50000
E = 800000
DN = 64
H = 64

_NC = 2
_NS = 16
_NW = _NC * _NS

_EPW = E // _NW
_GCH = 1000
_GSUB = (128, 128, 128, 128, 128, 128, 128, 104)


def _sc_gather_body(xw_hbm, row_hbm, g_hbm, idx_v, rows_v, sem):
    c = lax.axis_index("c")
    s = lax.axis_index("s")
    wid = s * _NC + c
    base = wid * _EPW

    def chunk(k, carry):
        off = base + k * _GCH
        pltpu.sync_copy(row_hbm.at[pl.ds(off, _GCH)], idx_v)
        descs = []
        o = 0
        for sz in _GSUB:
            descs.append(
                pltpu.async_copy(
                    xw_hbm.at[idx_v.at[pl.ds(o, sz)]],
                    rows_v.at[pl.ds(o, sz)],
                    sem,
                )
            )
            o += sz
        for d in descs:
            d.wait()
        pltpu.sync_copy(rows_v, g_hbm.at[pl.ds(off, _GCH)])
        return carry

    lax.fori_loop(0, _EPW // _GCH, chunk, 0)


def _sc_gather(xw, row):
    mesh = plsc.VectorSubcoreMesh(core_axis_name="c", subcore_axis_name="s")
    f = pl.kernel(
        _sc_gather_body,
        out_type=jax.ShapeDtypeStruct((E, H), jnp.float32),
        mesh=mesh,
        scratch_types=[
            pltpu.VMEM((_GCH,), jnp.int32),
            pltpu.VMEM((_GCH, H), jnp.float32),
            pltpu.SemaphoreType.DMA,
        ],
        compiler_params=pltpu.CompilerParams(use_tc_tiling_on_sc=False),
    )
    return f(xw, row)


_E_BLK = 3200


_HH = H // 2


def _edge_body(g_ref, ea_ref, wt_ref, wb_ref, b_ref, h0_ref, h1_ref,
               sh_ref, sh2_ref):
    z = (
        jnp.dot(g_ref[...], wt_ref[...], preferred_element_type=jnp.float32)
        + jnp.dot(ea_ref[...], wb_ref[...], preferred_element_type=jnp.float32)
        + b_ref[...]
    )
    h = jnp.maximum(z, 0.01 * z)
    h0_ref[...] = h[:, :_HH]
    h1_ref[...] = h[:, _HH:]

    @pl.when(pl.program_id(0) == 0)
    def _():
        sh_ref[...] = jnp.zeros_like(sh_ref)
        sh2_ref[...] = jnp.zeros_like(sh2_ref)

    sh_ref[...] += jnp.sum(h, axis=0, keepdims=True)
    sh2_ref[...] += jnp.sum(h * h, axis=0, keepdims=True)


def _tc_edge(g, edge_attr, w_top, w_bot, b1a):
    hspec = pl.BlockSpec((_E_BLK, _HH), lambda i: (i, 0))
    hshape = jax.ShapeDtypeStruct((E, _HH), jnp.float32)
    return pl.pallas_call(
        _edge_body,
        grid=(E // _E_BLK,),
        in_specs=[
            pl.BlockSpec((_E_BLK, H), lambda i: (i, 0)),
            pl.BlockSpec((_E_BLK, DN), lambda i: (i, 0)),
            pl.BlockSpec((DN, H), lambda i: (0, 0)),
            pl.BlockSpec((DN, H), lambda i: (0, 0)),
            pl.BlockSpec((1, H), lambda i: (0, 0)),
        ],
        out_specs=[
            hspec, hspec,
            pl.BlockSpec((1, H), lambda i: (0, 0)),
            pl.BlockSpec((1, H), lambda i: (0, 0)),
        ],
        out_shape=[
            hshape, hshape,
            jax.ShapeDtypeStruct((1, H), jnp.float32),
            jax.ShapeDtypeStruct((1, H), jnp.float32),
        ],
    )(g, edge_attr, w_top, w_bot, b1a)


_EPT = E // _NS
_IW = 125
_IR = 4
_SCH = _IW * _IR
_NST = 2
_NPT = N // _NS
_ZCH = 125
_DW = 8


def _sc_scatter_body(
    h0_hbm, h1_hbm, col2_hbm, z32_hbm,
    s0_hbm, s1_hbm,
    idx2_v, h_v, zb_v, sem, sacc,
):
    c = lax.axis_index("c")
    s = lax.axis_index("s")
    base = s * _EPT
    ibase = s * (_EPT // _SCH)

    def _scan(h_hbm):
        def chunk(k, carry):
            pltpu.sync_copy(col2_hbm.at[ibase + k], idx2_v)
            pltpu.sync_copy(h_hbm.at[pl.ds(base + k * _SCH, _SCH)], h_v)
            descs = [
                pltpu.async_copy(
                    h_v.at[pl.ds(j * _IW, _IW)],
                    sacc.at[idx2_v.at[j]],
                    sem,
                    add=True,
                )
                for j in range(_IR)
            ]
            for d in descs:
                d.wait()
            return carry

        lax.fori_loop(0, _EPT // _SCH, chunk, 0)

    def _dump(s_hbm):
        for k in range(_NPT // _ZCH):
            r = s * _NPT + k * _ZCH
            pltpu.sync_copy(sacc.at[pl.ds(r, _ZCH)], zb_v)
            pltpu.sync_copy(zb_v, s_hbm.at[pl.ds(r, _ZCH)])

    for k in range(_NPT // _ZCH):
        r = s * _NPT + k * _ZCH
        pltpu.sync_copy(z32_hbm.at[pl.ds(r, _ZCH)], zb_v)
        pltpu.sync_copy(zb_v, sacc.at[pl.ds(r, _ZCH)])

    plsc.subcore_barrier()

    @pl.when(c == 0)
    def _():
        _scan(h0_hbm)

    @pl.when(c == 1)
    def _():
        _scan(h1_hbm)

    plsc.subcore_barrier()

    @pl.when(c == 0)
    def _():
        _dump(s0_hbm)

    @pl.when(c == 1)
    def _():
        _dump(s1_hbm)


def _sc_scatter(h0, h1, col2):
    mesh = plsc.VectorSubcoreMesh(core_axis_name="c", subcore_axis_name="s")
    z32 = jnp.zeros((N, _HH), jnp.float32)
    sshape = jax.ShapeDtypeStruct((N, _HH), jnp.float32)
    f = pl.kernel(
        _sc_scatter_body,
        out_type=(sshape, sshape),
        mesh=mesh,
        scratch_types=[
            pltpu.VMEM((_IR, _IW), jnp.int32),
            pltpu.VMEM((_SCH, _HH), jnp.float32),
            pltpu.VMEM((_ZCH, _HH), jnp.float32),
            pltpu.SemaphoreType.DMA,
            pltpu.VMEM_SHARED((N, _HH), jnp.float32),
        ],
        compiler_params=pltpu.CompilerParams(use_tc_tiling_on_sc=False),
    )
    return f(h0, h1, col2, z32)


def _sc_deg_body(
    col2_hbm, zd_hbm, ones_hbm,
    d0_hbm, d1_hbm,
    idx2_v, ones_v, zbd_v, sem, dacc,
):
    c = lax.axis_index("c")
    s = lax.axis_index("s")

    for k in range(_NPT // _ZCH):
        r = s * _NPT + k * _ZCH
        pltpu.sync_copy(zd_hbm.at[pl.ds(r, _ZCH)], zbd_v)
        pltpu.sync_copy(zbd_v, dacc.at[pl.ds(r, _ZCH)])

    pltpu.sync_copy(ones_hbm, ones_v)
    plsc.subcore_barrier()

    wid = s * _NC + c
    ibase = wid * (_EPW // _SCH)

    def chunk(k, carry):
        pltpu.sync_copy(col2_hbm.at[ibase + k], idx2_v)
        descs = [
            pltpu.async_copy(ones_v, dacc.at[idx2_v.at[j]], sem, add=True)
            for j in range(_IR)
        ]
        for d in descs:
            d.wait()
        return carry

    lax.fori_loop(0, _EPW // _SCH, chunk, 0)
    plsc.subcore_barrier()

    def _dump(d_hbm):
        for k in range(_NPT // _ZCH):
            r = s * _NPT + k * _ZCH
            pltpu.sync_copy(dacc.at[pl.ds(r, _ZCH)], zbd_v)
            pltpu.sync_copy(zbd_v, d_hbm.at[pl.ds(r, _ZCH)])

    @pl.when(c == 0)
    def _():
        _dump(d0_hbm)

    @pl.when(c == 1)
    def _():
        _dump(d1_hbm)


def _sc_deg(col2):
    mesh = plsc.VectorSubcoreMesh(core_axis_name="c", subcore_axis_name="s")
    zd = jnp.zeros((N, _DW), jnp.float32)
    ones = jnp.ones((_IW, _DW), jnp.float32)
    f = pl.kernel(
        _sc_deg_body,
        out_type=(
            jax.ShapeDtypeStruct((N, _DW), jnp.float32),
            jax.ShapeDtypeStruct((N, _DW), jnp.float32),
        ),
        mesh=mesh,
        scratch_types=[
            pltpu.VMEM((_IR, _IW), jnp.int32),
            pltpu.VMEM((_IW, _DW), jnp.float32),
            pltpu.VMEM((_ZCH, _DW), jnp.float32),
            pltpu.SemaphoreType.DMA,
            pltpu.VMEM_SHARED((N, _DW), jnp.float32),
        ],
        compiler_params=pltpu.CompilerParams(use_tc_tiling_on_sc=False),
    )
    return f(col2, zd, ones)


_N_BLK = 2000


def _node1_body(
    s0_ref, s1_ref, d0_ref, d1_ref, x_ref, sh_ref, sh2_ref,
    g1_ref, be1_ref, w1b_ref, b1b_ref, w2at_ref, w2ab_ref, b2a_ref,
    t_ref, st_ref, st2_ref,
):
    mu = sh_ref[...] / E
    var = sh2_ref[...] / E - mu * mu
    winv = g1_ref[...] * lax.rsqrt(var + 1e-5)
    cvec = (
        jnp.dot(be1_ref[...] - winv * mu, w1b_ref[...],
                preferred_element_type=jnp.float32)
        + b1b_ref[...]
    )
    sb = jnp.concatenate([s0_ref[...], s1_ref[...]], axis=1)
    deg = (d0_ref[...] + d1_ref[...])[:, :1]
    agg = (
        jnp.dot(winv * sb, w1b_ref[...], preferred_element_type=jnp.float32)
        + deg * cvec
    )
    t = (
        jnp.dot(x_ref[...], w2at_ref[...], preferred_element_type=jnp.float32)
        + jnp.dot(agg, w2ab_ref[...], preferred_element_type=jnp.float32)
        + b2a_ref[...]
    )
    t = jnp.maximum(t, 0.01 * t)
    t_ref[...] = t

    @pl.when(pl.program_id(0) == 0)
    def _():
        st_ref[...] = jnp.zeros_like(st_ref)
        st2_ref[...] = jnp.zeros_like(st2_ref)

    st_ref[...] += jnp.sum(t, axis=0, keepdims=True)
    st2_ref[...] += jnp.sum(t * t, axis=0, keepdims=True)


def _tc_node1(s0, s1, d0, d1, x, sh, sh2, g1, be1, w1b, b1b,
              w2at, w2ab, b2a):
    sspec = pl.BlockSpec((_N_BLK, _HH), lambda i: (i, 0))
    return pl.pallas_call(
        _node1_body,
        grid=(N // _N_BLK,),
        in_specs=[
            sspec, sspec,
            pl.BlockSpec((_N_BLK, _DW), lambda i: (i, 0)),
            pl.BlockSpec((_N_BLK, _DW), lambda i: (i, 0)),
            pl.BlockSpec((_N_BLK, DN), lambda i: (i, 0)),
            pl.BlockSpec((1, H), lambda i: (0, 0)),
            pl.BlockSpec((1, H), lambda i: (0, 0)),
            pl.BlockSpec((1, H), lambda i: (0, 0)),
            pl.BlockSpec((1, H), lambda i: (0, 0)),
            pl.BlockSpec((H, H), lambda i: (0, 0)),
            pl.BlockSpec((1, H), lambda i: (0, 0)),
            pl.BlockSpec((DN, H), lambda i: (0, 0)),
            pl.BlockSpec((H, H), lambda i: (0, 0)),
            pl.BlockSpec((1, H), lambda i: (0, 0)),
        ],
        out_specs=[
            pl.BlockSpec((_N_BLK, H), lambda i: (i, 0)),
            pl.BlockSpec((1, H), lambda i: (0, 0)),
            pl.BlockSpec((1, H), lambda i: (0, 0)),
        ],
        out_shape=[
            jax.ShapeDtypeStruct((N, H), jnp.float32),
            jax.ShapeDtypeStruct((1, H), jnp.float32),
            jax.ShapeDtypeStruct((1, H), jnp.float32),
        ],
    )(s0, s1, d0, d1, x, sh, sh2, g1, be1, w1b, b1b, w2at, w2ab, b2a)


def _node2_body(t_ref, st_ref, st2_ref, g2_ref, be2_ref, w2b_ref, b2b_ref, o_ref):
    mu = st_ref[...] / N
    var = st2_ref[...] / N - mu * mu
    winv = g2_ref[...] * lax.rsqrt(var + 1e-5)
    y = winv * (t_ref[...] - mu) + be2_ref[...]
    o_ref[...] = (
        jnp.dot(y, w2b_ref[...], preferred_element_type=jnp.float32)
        + b2b_ref[...]
    )


def _tc_node2(t, st, st2, g2, be2, w2b, b2b):
    return pl.pallas_call(
        _node2_body,
        grid=(N // _N_BLK,),
        in_specs=[
            pl.BlockSpec((_N_BLK, H), lambda i: (i, 0)),
            pl.BlockSpec((1, H), lambda i: (0, 0)),
            pl.BlockSpec((1, H), lambda i: (0, 0)),
            pl.BlockSpec((1, H), lambda i: (0, 0)),
            pl.BlockSpec((1, H), lambda i: (0, 0)),
            pl.BlockSpec((H, DN), lambda i: (0, 0)),
            pl.BlockSpec((1, DN), lambda i: (0, 0)),
        ],
        out_specs=pl.BlockSpec((_N_BLK, DN), lambda i: (i, 0)),
        out_shape=jax.ShapeDtypeStruct((N, DN), jnp.float32),
    )(t, st, st2, g2, be2, w2b, b2b)


def kernel(x, edge_index, edge_attr, u, batch, W1a, b1a, g1, be1, W1b, b1b,
           W2a, b2a, g2, be2, W2b, b2b):
    row = edge_index[0]
    col2 = edge_index[1].reshape(E // _SCH, _IR, _IW)
    r1 = lambda v: v.reshape(1, -1)

    d0, d1 = _sc_deg(col2)
    g = _sc_gather(x, row)
    h0, h1, sh, sh2 = _tc_edge(g, edge_attr, W1a[:DN], W1a[DN:], r1(b1a))
    s0, s1 = _sc_scatter(h0, h1, col2)
    t, st, st2 = _tc_node1(
        s0, s1, d0, d1, x, sh, sh2, r1(g1), r1(be1), W1b, r1(b1b),
        W2a[:DN], W2a[DN:], r1(b2a),
    )
    return _tc_node2(t, st, st2, r1(g2), r1(be2), W2b, r1(b2b))
